# Initial kernel scaffold; baseline (speedup 1.0000x reference)
import jax
import jax.numpy as jnp
from jax.experimental import pallas as pl


def kernel(taxPayer_feats, person_feats, item_feats, trans_adj_list, P_company, P_person, P_item, W_PCC):
    raise NotImplementedError("write your pallas kernel here")


# speedup vs baseline: 12.7631x; 12.7631x over previous
import functools

import jax
import jax.numpy as jnp
from jax import lax
from jax.experimental import pallas as pl
from jax.experimental.pallas import tpu as pltpu
from jax.experimental.pallas import tpu_sc as plsc

_N = 10000
_E = 320000
_D = 128
_NHEADS = 4
_ALPHA = 0.2

_NC = 2
_NS = 16
_NW = _NC * _NS
_EPW = _E // _NW
_C = 80
_NCHUNK = _EPW // _C

_BN = 2000


def _tc_proj(xt, xp, pc, pp, w):
    def body(xt_ref, xp_ref, pc_ref, pp_ref, w_ref, g0_ref, g1_ref, g2_ref):
        f32 = jnp.float32
        a0 = jnp.concatenate(
            [jnp.dot(pc_ref[h], w_ref[h, 0:32, :], preferred_element_type=f32)
             for h in range(_NHEADS)], axis=1)
        a1 = jnp.concatenate(
            [jnp.dot(pc_ref[h], w_ref[h, 32:64, :], preferred_element_type=f32)
             for h in range(_NHEADS)], axis=1)
        a2 = jnp.concatenate(
            [jnp.dot(pp_ref[h], w_ref[h, 64:96, :], preferred_element_type=f32)
             for h in range(_NHEADS)], axis=1)
        xt_blk = xt_ref[...]
        g0_ref[...] = jnp.dot(xt_blk, a0, preferred_element_type=f32)
        g1_ref[...] = jnp.dot(xt_blk, a1, preferred_element_type=f32)
        g2_ref[...] = jnp.dot(xp_ref[...], a2, preferred_element_type=f32)

    out = jax.ShapeDtypeStruct((_N, _D), jnp.float32)
    return pl.pallas_call(
        body,
        grid=(_N // _BN,),
        in_specs=[
            pl.BlockSpec((_BN, _D), lambda i: (i, 0)),
            pl.BlockSpec((_BN, _D), lambda i: (i, 0)),
            pl.BlockSpec((_NHEADS, _D, 32), lambda i: (0, 0, 0)),
            pl.BlockSpec((_NHEADS, _D, 32), lambda i: (0, 0, 0)),
            pl.BlockSpec((_NHEADS, 96, 32), lambda i: (0, 0, 0)),
        ],
        out_specs=[
            pl.BlockSpec((_BN, _D), lambda i: (i, 0)),
            pl.BlockSpec((_BN, _D), lambda i: (i, 0)),
            pl.BlockSpec((_BN, _D), lambda i: (i, 0)),
        ],
        out_shape=[out, out, out],
    )(xt, xp, pc, pp, w)


def _sc_gather(g0, g1, g2, i0, i1, i2):
    mesh = plsc.VectorSubcoreMesh(core_axis_name="c", subcore_axis_name="s")

    @functools.partial(
        pl.kernel,
        out_type=jax.ShapeDtypeStruct((_E, _D), jnp.float32),
        mesh=mesh,
        scratch_types=[
            pltpu.VMEM((_C,), jnp.int32),
            pltpu.VMEM((_C,), jnp.int32),
            pltpu.VMEM((_C,), jnp.int32),
            pltpu.VMEM((3, _C, _D), jnp.float32),
            pltpu.SemaphoreType.DMA,
        ],
    )
    def body(g0_hbm, g1_hbm, g2_hbm, i0_hbm, i1_hbm, i2_hbm, out_hbm,
             idx0_v, idx1_v, idx2_v, rows_v, sem):
        wid = lax.axis_index("s") * _NC + lax.axis_index("c")
        tables = (g0_hbm, g1_hbm, g2_hbm)
        idx_hbms = (i0_hbm, i1_hbm, i2_hbm)
        idx_bufs = (idx0_v, idx1_v, idx2_v)

        def chunk(g, carry):
            off = pl.multiple_of(wid * _EPW + g * _C, 8)
            for k in range(3):
                pltpu.sync_copy(idx_hbms[k].at[pl.ds(off, _C)], idx_bufs[k])
            cps = [pltpu.async_copy(tables[k].at[idx_bufs[k]], rows_v.at[k], sem)
                   for k in range(3)]
            for cp in cps:
                cp.wait()

            def edge(e, carry2):
                for j in range(_D // 16):
                    s = pl.ds(j * 16, 16)
                    x = rows_v[0, e, s] + rows_v[1, e, s] + rows_v[2, e, s]
                    rows_v[0, e, s] = jnp.maximum(x, x * _ALPHA)
                return carry2

            lax.fori_loop(0, _C, edge, 0)
            pltpu.sync_copy(rows_v.at[0], out_hbm.at[pl.ds(off, _C)])
            return carry

        lax.fori_loop(0, _NCHUNK, chunk, 0)

    return body(g0, g1, g2, i0, i1, i2)


def kernel(taxPayer_feats, person_feats, item_feats, trans_adj_list,
           P_company, P_person, P_item, W_PCC):
    del item_feats, P_item
    g0, g1, g2 = _tc_proj(taxPayer_feats, person_feats, P_company, P_person, W_PCC)
    return _sc_gather(g0, g1, g2,
                      trans_adj_list[0], trans_adj_list[1], trans_adj_list[2])


# speedup vs baseline: 29.4219x; 2.3052x over previous
import functools

import jax
import jax.numpy as jnp
from jax import lax
from jax.experimental import pallas as pl
from jax.experimental.pallas import tpu as pltpu
from jax.experimental.pallas import tpu_sc as plsc

_N = 10000
_E = 320000
_D = 128
_NHEADS = 4
_ALPHA = 0.2

_NC = 2
_NS = 16
_NW = _NC * _NS
_EPW = _E // _NW
_C = 80
_NCHUNK = _EPW // _C

_BN = 2000


def _tc_proj(xt, xp, pc, pp, w):
    def body(xt_ref, xp_ref, pc_ref, pp_ref, w_ref, g0_ref, g1_ref, g2_ref):
        f32 = jnp.float32
        a0 = jnp.concatenate(
            [jnp.dot(pc_ref[h], w_ref[h, 0:32, :], preferred_element_type=f32)
             for h in range(_NHEADS)], axis=1)
        a1 = jnp.concatenate(
            [jnp.dot(pc_ref[h], w_ref[h, 32:64, :], preferred_element_type=f32)
             for h in range(_NHEADS)], axis=1)
        a2 = jnp.concatenate(
            [jnp.dot(pp_ref[h], w_ref[h, 64:96, :], preferred_element_type=f32)
             for h in range(_NHEADS)], axis=1)
        xt_blk = xt_ref[...]
        g0_ref[...] = jnp.dot(xt_blk, a0, preferred_element_type=f32)
        g1_ref[...] = jnp.dot(xt_blk, a1, preferred_element_type=f32)
        g2_ref[...] = jnp.dot(xp_ref[...], a2, preferred_element_type=f32)

    out = jax.ShapeDtypeStruct((_N, _D), jnp.float32)
    return pl.pallas_call(
        body,
        grid=(_N // _BN,),
        in_specs=[
            pl.BlockSpec((_BN, _D), lambda i: (i, 0)),
            pl.BlockSpec((_BN, _D), lambda i: (i, 0)),
            pl.BlockSpec((_NHEADS, _D, 32), lambda i: (0, 0, 0)),
            pl.BlockSpec((_NHEADS, _D, 32), lambda i: (0, 0, 0)),
            pl.BlockSpec((_NHEADS, 96, 32), lambda i: (0, 0, 0)),
        ],
        out_specs=[
            pl.BlockSpec((_BN, _D), lambda i: (i, 0)),
            pl.BlockSpec((_BN, _D), lambda i: (i, 0)),
            pl.BlockSpec((_BN, _D), lambda i: (i, 0)),
        ],
        out_shape=[out, out, out],
    )(xt, xp, pc, pp, w)


def _sc_gather(g0, g1, g2, i0, i1, i2):
    mesh = plsc.VectorSubcoreMesh(core_axis_name="c", subcore_axis_name="s")

    @functools.partial(
        pl.kernel,
        out_type=jax.ShapeDtypeStruct((_E, _D), jnp.float32),
        mesh=mesh,
        scratch_types=[
            [pltpu.VMEM((_C,), jnp.int32)] * 3,
            [pltpu.VMEM((_C,), jnp.int32)] * 3,
            pltpu.VMEM((3, _C, _D), jnp.float32),
            pltpu.VMEM((3, _C, _D), jnp.float32),
            pltpu.VMEM((_C, _D), jnp.float32),
            pltpu.VMEM((_C, _D), jnp.float32),
            pltpu.SemaphoreType.DMA,
            pltpu.SemaphoreType.DMA,
            pltpu.SemaphoreType.DMA,
            pltpu.SemaphoreType.DMA,
            pltpu.SemaphoreType.DMA,
            pltpu.SemaphoreType.DMA,
        ],
    )
    def body(g0_hbm, g1_hbm, g2_hbm, i0_hbm, i1_hbm, i2_hbm, out_hbm,
             idx_a, idx_b, rows_a, rows_b, obuf_a, obuf_b,
             sem_ia, sem_ib, sem_ga, sem_gb, sem_oa, sem_ob):
        wid = lax.axis_index("s") * _NC + lax.axis_index("c")
        base = pl.multiple_of(wid * _EPW, 8)
        tables = (g0_hbm, g1_hbm, g2_hbm)
        idx_hbms = (i0_hbm, i1_hbm, i2_hbm)

        def fire_idx(g, bufs, sem):
            off = pl.multiple_of(base + g * _C, 8)
            for k in range(3):
                pltpu.async_copy(idx_hbms[k].at[pl.ds(off, _C)], bufs[k], sem)

        def drain_idx(bufs, sem):
            for k in range(3):
                pltpu.make_async_copy(
                    idx_hbms[k].at[pl.ds(base, _C)], bufs[k], sem).wait()

        def fire_gathers(bufs, rows, sem):
            for k in range(3):
                pltpu.async_copy(tables[k].at[bufs[k]], rows.at[k], sem)

        def drain_gathers(bufs, rows, sem):
            for k in range(3):
                pltpu.make_async_copy(tables[k].at[bufs[k]], rows.at[k], sem
                                      ).wait()

        def compute(rows, obuf):
            def edge(e, carry):
                for j in range(_D // 16):
                    s = pl.ds(j * 16, 16)
                    x = rows[0, e, s] + rows[1, e, s] + rows[2, e, s]
                    obuf[e, s] = jnp.maximum(x, x * _ALPHA)
                return carry

            lax.fori_loop(0, _C, edge, 0)

        def fire_out(g, obuf, sem):
            off = pl.multiple_of(base + g * _C, 8)
            pltpu.async_copy(obuf, out_hbm.at[pl.ds(off, _C)], sem)

        def drain_out(obuf, sem):
            pltpu.make_async_copy(obuf, out_hbm.at[pl.ds(base, _C)], sem).wait()

        fire_idx(0, idx_a, sem_ia)
        drain_idx(idx_a, sem_ia)
        fire_gathers(idx_a, rows_a, sem_ga)
        fire_idx(1, idx_b, sem_ib)
        drain_idx(idx_b, sem_ib)
        fire_gathers(idx_b, rows_b, sem_gb)

        def pair(p, carry):
            g = p * 2
            drain_gathers(idx_a, rows_a, sem_ga)
            fire_idx(g + 2, idx_a, sem_ia)
            pl.when(p >= 1)(lambda: drain_out(obuf_a, sem_oa))
            compute(rows_a, obuf_a)
            fire_out(g, obuf_a, sem_oa)
            drain_idx(idx_a, sem_ia)
            fire_gathers(idx_a, rows_a, sem_ga)
            drain_gathers(idx_b, rows_b, sem_gb)
            pl.when(g + 3 < _NCHUNK)(lambda: fire_idx(g + 3, idx_b, sem_ib))
            pl.when(p >= 1)(lambda: drain_out(obuf_b, sem_ob))
            compute(rows_b, obuf_b)
            fire_out(g + 1, obuf_b, sem_ob)

            def _next_b():
                drain_idx(idx_b, sem_ib)
                fire_gathers(idx_b, rows_b, sem_gb)

            pl.when(g + 3 < _NCHUNK)(_next_b)
            return carry

        lax.fori_loop(0, (_NCHUNK - 1) // 2, pair, 0)

        drain_gathers(idx_a, rows_a, sem_ga)
        drain_out(obuf_a, sem_oa)
        compute(rows_a, obuf_a)
        fire_out(_NCHUNK - 1, obuf_a, sem_oa)
        drain_out(obuf_a, sem_oa)
        drain_out(obuf_b, sem_ob)

    return body(g0, g1, g2, i0, i1, i2)


def kernel(taxPayer_feats, person_feats, item_feats, trans_adj_list,
           P_company, P_person, P_item, W_PCC):
    del item_feats, P_item
    g0, g1, g2 = _tc_proj(taxPayer_feats, person_feats, P_company, P_person, W_PCC)
    return _sc_gather(g0, g1, g2,
                      trans_adj_list[0], trans_adj_list[1], trans_adj_list[2])
